# baseline (device time: 24391 ns/iter reference)
import jax
import jax.numpy as jnp
from jax import lax
from jax.experimental import pallas as pl
from jax.experimental.pallas import tpu as pltpu

N_DEV = 4
N_TOK = 1024
D_IN = 512
D_OUT = 1024
N_EXP = 16
E_LOCAL = 4
CAP = 51
PAD_E = 64
BLK = E_LOCAL * PAD_E
GATH = N_DEV * BLK
HALF = BLK // 2
N_SEG = 2
SEG = HALF // N_SEG
ROWS = N_TOK // N_DEV
N_STEP = N_DEV - 1


def kernel(x, router_W, route_idx, expert_W):
    del router_W

    def body(x_ref, ridx_ref, ew_hbm, out_ref,
             gslot_ref, yall_ref, s_ref, ew_ref, ew_sems,
             sendR, recvR, sendL, recvL):
        my = lax.axis_index("i")
        left = (my - 1) % N_DEV
        right = (my + 1) % N_DEV

        ew_copies = []
        for j in range(E_LOCAL):
            c = pltpu.make_async_copy(
                ew_hbm.at[j], ew_ref.at[j], ew_sems.at[j])
            c.start()
            ew_copies.append(c)

        barrier = pltpu.get_barrier_semaphore()
        for nbr in [left, right]:
            pl.semaphore_signal(
                barrier, inc=1,
                device_id=(nbr,), device_id_type=pl.DeviceIdType.MESH,
            )
        pl.semaphore_wait(barrier, 2)

        ridx = ridx_ref[:, :]

        def inclusive_cumsum(mat_i16, ncol):
            cum = mat_i16
            sh = 1
            while sh < N_TOK:
                cum = cum + jnp.concatenate(
                    [jnp.zeros((sh, ncol), jnp.int16), cum[:-sh]], axis=0
                )
                sh *= 2
            return cum.astype(jnp.float32)

        eids4 = my * E_LOCAL + lax.broadcasted_iota(jnp.int32, (1, E_LOCAL), 1)
        onehot4_i = (ridx == eids4).astype(jnp.int16)
        cum4 = inclusive_cumsum(onehot4_i, E_LOCAL)
        onehot4 = onehot4_i.astype(jnp.float32)
        keep4 = onehot4 * (cum4 <= CAP).astype(jnp.float32)
        j_off = (PAD_E * lax.broadcasted_iota(jnp.int32, (1, E_LOCAL), 1)
                 ).astype(jnp.float32)
        val = jnp.sum(keep4 * (j_off + cum4 - 1.0), axis=1, keepdims=True)
        keptloc = jnp.sum(keep4, axis=1, keepdims=True)
        slot_loc = jnp.where(keptloc > 0.0, val, -1.0)

        def make(h, g, ring):
            if ring == "R":
                snd, rcv, tgt, hoff = sendR, recvR, right, 0
                origin = (my - h) % N_DEV
            else:
                snd, rcv, tgt, hoff = sendL, recvL, left, HALF
                origin = (my + h) % N_DEV
            sl = pl.ds(origin * BLK + hoff + g * SEG, SEG)
            return pltpu.make_async_remote_copy(
                src_ref=yall_ref.at[sl, :], dst_ref=yall_ref.at[sl, :],
                send_sem=snd.at[h, g], recv_sem=rcv.at[h, g],
                device_id=(tgt,), device_id_type=pl.DeviceIdType.MESH,
            )

        my_base = my * BLK
        slot_lane = lax.broadcasted_iota(
            jnp.int32, (1, BLK), 1).astype(jnp.float32)
        g_t = (slot_loc == slot_lane).astype(jnp.float32)
        x_c = lax.dot_general(
            g_t, x_ref[:, :], (((0,), (0,)), ((), ())),
            preferred_element_type=jnp.float32,
        )
        live = {}
        hop0 = {0: (0, "R"), 1: (1, "R"), 2: (0, "L"), 3: (1, "L")}
        for j in range(E_LOCAL):
            ew_copies[j].wait()
            yall_ref[pl.ds(my_base + j * PAD_E, PAD_E), :] = jnp.dot(
                x_c[j * PAD_E:(j + 1) * PAD_E, :], ew_ref[j],
                preferred_element_type=jnp.float32,
            ).astype(jnp.bfloat16)
            g, ring = hop0[j]
            r = make(0, g, ring)
            r.start()
            live[(0, g, ring)] = r

        eids = lax.broadcasted_iota(jnp.int32, (1, N_EXP), 1)
        onehot_i = (ridx == eids).astype(jnp.int16)
        cum = inclusive_cumsum(onehot_i, N_EXP)
        onehot = onehot_i.astype(jnp.float32)
        rank = jnp.sum(onehot * cum, axis=1, keepdims=True)
        kept = jnp.sum(onehot * (cum <= CAP).astype(jnp.float32),
                       axis=1, keepdims=True)
        eloc = (ridx % E_LOCAL).astype(jnp.float32)
        chip = (ridx // E_LOCAL).astype(jnp.float32)
        gslot = jnp.where(
            kept > 0.0, chip * BLK + eloc * PAD_E + rank - 1.0, -1.0
        )
        gslot_ref[:, :] = gslot

        gslot_q = gslot_ref[pl.ds(my * ROWS, ROWS), :]
        lane_all = lax.broadcasted_iota(
            jnp.int32, (1, GATH), 1).astype(jnp.float32)
        s_ref[:, :] = (gslot_q == lane_all).astype(jnp.bfloat16)

        for h in range(N_STEP):
            for g in range(N_SEG):
                for ring in ("R", "L"):
                    live[(h, g, ring)].wait()
                    if h + 1 < N_STEP:
                        r = make(h + 1, g, ring)
                        r.start()
                        live[(h + 1, g, ring)] = r

        out_ref[:, :] = jnp.dot(s_ref[:, :], yall_ref[:, :],
                                preferred_element_type=jnp.float32)

    return pl.pallas_call(
        body,
        out_shape=jax.ShapeDtypeStruct((ROWS, D_OUT), jnp.float32),
        in_specs=[
            pl.BlockSpec(memory_space=pltpu.VMEM),
            pl.BlockSpec(memory_space=pltpu.VMEM),
            pl.BlockSpec(memory_space=pltpu.HBM),
        ],
        out_specs=pl.BlockSpec(memory_space=pltpu.VMEM),
        scratch_shapes=[
            pltpu.VMEM((N_TOK, 1), jnp.float32),
            pltpu.VMEM((GATH, D_OUT), jnp.bfloat16),
            pltpu.VMEM((ROWS, GATH), jnp.bfloat16),
            pltpu.VMEM((E_LOCAL, D_IN, D_OUT), jnp.float32),
            pltpu.SemaphoreType.DMA((E_LOCAL,)),
            pltpu.SemaphoreType.DMA((N_STEP, N_SEG)),
            pltpu.SemaphoreType.DMA((N_STEP, N_SEG)),
            pltpu.SemaphoreType.DMA((N_STEP, N_SEG)),
            pltpu.SemaphoreType.DMA((N_STEP, N_SEG)),
        ],
        compiler_params=pltpu.CompilerParams(collective_id=0),
    )(x, route_idx, expert_W)


# device time: 22901 ns/iter; 1.0651x vs baseline; 1.0651x over previous
import jax
import jax.numpy as jnp
from jax import lax
from jax.experimental import pallas as pl
from jax.experimental.pallas import tpu as pltpu

N_DEV = 4
N_TOK = 1024
D_IN = 512
D_OUT = 1024
N_EXP = 16
E_LOCAL = 4
CAP = 51
PAD_E = 64
BLK = E_LOCAL * PAD_E
GATH = N_DEV * BLK
HALF = BLK // 2
N_SEG = 4
SEG = HALF // N_SEG
ROWS = N_TOK // N_DEV
N_STEP = N_DEV - 1


def kernel(x, router_W, route_idx, expert_W):
    del router_W

    def body(x_ref, ridx_ref, ew_hbm, out_ref,
             gslot_ref, yall_ref, s_ref, ew_ref, ew_sems,
             sendR, recvR, sendL, recvL):
        my = lax.axis_index("i")
        left = (my - 1) % N_DEV
        right = (my + 1) % N_DEV

        ew_copies = []
        for j in range(E_LOCAL):
            c = pltpu.make_async_copy(
                ew_hbm.at[j], ew_ref.at[j], ew_sems.at[j])
            c.start()
            ew_copies.append(c)

        barrier = pltpu.get_barrier_semaphore()
        for nbr in [left, right]:
            pl.semaphore_signal(
                barrier, inc=1,
                device_id=(nbr,), device_id_type=pl.DeviceIdType.MESH,
            )
        pl.semaphore_wait(barrier, 2)

        ridx = ridx_ref[:, :]

        def inclusive_cumsum(mat_i16, ncol):
            cum = mat_i16
            sh = 1
            while sh < N_TOK:
                cum = cum + jnp.concatenate(
                    [jnp.zeros((sh, ncol), jnp.int16), cum[:-sh]], axis=0
                )
                sh *= 2
            return cum.astype(jnp.float32)

        eids4 = my * E_LOCAL + lax.broadcasted_iota(jnp.int32, (1, E_LOCAL), 1)
        onehot4_i = (ridx == eids4).astype(jnp.int16)
        cum4 = inclusive_cumsum(onehot4_i, E_LOCAL)
        onehot4 = onehot4_i.astype(jnp.float32)
        keep4 = onehot4 * (cum4 <= CAP).astype(jnp.float32)
        j_off = (PAD_E * lax.broadcasted_iota(jnp.int32, (1, E_LOCAL), 1)
                 ).astype(jnp.float32)
        val = jnp.sum(keep4 * (j_off + cum4 - 1.0), axis=1, keepdims=True)
        keptloc = jnp.sum(keep4, axis=1, keepdims=True)
        slot_loc = jnp.where(keptloc > 0.0, val, -1.0)

        def make(h, g, ring):
            if ring == "R":
                snd, rcv, tgt, hoff = sendR, recvR, right, 0
                origin = (my - h) % N_DEV
            else:
                snd, rcv, tgt, hoff = sendL, recvL, left, HALF
                origin = (my + h) % N_DEV
            sl = pl.ds(origin * BLK + hoff + g * SEG, SEG)
            return pltpu.make_async_remote_copy(
                src_ref=yall_ref.at[sl, :], dst_ref=yall_ref.at[sl, :],
                send_sem=snd.at[h, g], recv_sem=rcv.at[h, g],
                device_id=(tgt,), device_id_type=pl.DeviceIdType.MESH,
            )

        my_base = my * BLK
        slot_lane = lax.broadcasted_iota(
            jnp.int32, (1, BLK), 1).astype(jnp.float32)
        g_t = (slot_loc == slot_lane).astype(jnp.float32)
        x_c = lax.dot_general(
            g_t, x_ref[:, :], (((0,), (0,)), ((), ())),
            preferred_element_type=jnp.float32,
        )
        live = {}
        hop0 = {
            0: [(0, "R"), (1, "R")],
            1: [(2, "R"), (3, "R")],
            2: [(0, "L"), (1, "L")],
            3: [(2, "L"), (3, "L")],
        }
        for j in range(E_LOCAL):
            ew_copies[j].wait()
            yall_ref[pl.ds(my_base + j * PAD_E, PAD_E), :] = jnp.dot(
                x_c[j * PAD_E:(j + 1) * PAD_E, :], ew_ref[j],
                preferred_element_type=jnp.float32,
            ).astype(jnp.bfloat16)
            for g, ring in hop0[j]:
                r = make(0, g, ring)
                r.start()
                live[(0, g, ring)] = r

        eids = lax.broadcasted_iota(jnp.int32, (1, N_EXP), 1)
        onehot_i = (ridx == eids).astype(jnp.int16)
        cum = inclusive_cumsum(onehot_i, N_EXP)
        onehot = onehot_i.astype(jnp.float32)
        rank = jnp.sum(onehot * cum, axis=1, keepdims=True)
        kept = jnp.sum(onehot * (cum <= CAP).astype(jnp.float32),
                       axis=1, keepdims=True)
        eloc = (ridx % E_LOCAL).astype(jnp.float32)
        chip = (ridx // E_LOCAL).astype(jnp.float32)
        gslot = jnp.where(
            kept > 0.0, chip * BLK + eloc * PAD_E + rank - 1.0, -1.0
        )
        gslot_ref[:, :] = gslot

        gslot_q = gslot_ref[pl.ds(my * ROWS, ROWS), :]
        lane_all = lax.broadcasted_iota(
            jnp.int32, (1, GATH), 1).astype(jnp.float32)
        s_ref[:, :] = (gslot_q == lane_all).astype(jnp.bfloat16)

        for h in range(N_STEP):
            for g in range(N_SEG):
                for ring in ("R", "L"):
                    live[(h, g, ring)].wait()
                    if h + 1 < N_STEP:
                        r = make(h + 1, g, ring)
                        r.start()
                        live[(h + 1, g, ring)] = r

        out_ref[:, :] = jnp.dot(s_ref[:, :], yall_ref[:, :],
                                preferred_element_type=jnp.float32)

    return pl.pallas_call(
        body,
        out_shape=jax.ShapeDtypeStruct((ROWS, D_OUT), jnp.float32),
        in_specs=[
            pl.BlockSpec(memory_space=pltpu.VMEM),
            pl.BlockSpec(memory_space=pltpu.VMEM),
            pl.BlockSpec(memory_space=pltpu.HBM),
        ],
        out_specs=pl.BlockSpec(memory_space=pltpu.VMEM),
        scratch_shapes=[
            pltpu.VMEM((N_TOK, 1), jnp.float32),
            pltpu.VMEM((GATH, D_OUT), jnp.bfloat16),
            pltpu.VMEM((ROWS, GATH), jnp.bfloat16),
            pltpu.VMEM((E_LOCAL, D_IN, D_OUT), jnp.float32),
            pltpu.SemaphoreType.DMA((E_LOCAL,)),
            pltpu.SemaphoreType.DMA((N_STEP, N_SEG)),
            pltpu.SemaphoreType.DMA((N_STEP, N_SEG)),
            pltpu.SemaphoreType.DMA((N_STEP, N_SEG)),
            pltpu.SemaphoreType.DMA((N_STEP, N_SEG)),
        ],
        compiler_params=pltpu.CompilerParams(collective_id=0),
    )(x, route_idx, expert_W)
